# baseline (speedup 1.0000x reference)
import functools

import jax
import jax.numpy as jnp
from jax import lax
from jax.experimental import pallas as pl
from jax.experimental.pallas import tpu as pltpu
from jax.experimental.pallas import tpu_sc as plsc

B, H, W = 64, 32, 32
HW = H * W
E = B * HW
BITS = 8
NODES = 255
NC, NS, L = 2, 16, 16
NW = NC * NS
EPW = E // NW
VPW = EPW // L
ROWS = EPW // 128


@functools.partial(
    pl.kernel,
    out_type=jax.ShapeDtypeStruct((BITS, NW, ROWS, 128), jnp.float32),
    mesh=plsc.VectorSubcoreMesh(
        core_axis_name="c", subcore_axis_name="s", num_cores=NC,
        num_subcores=NS),
    scratch_types=[
        pltpu.VMEM((EPW,), jnp.float32),
        pltpu.VMEM((BITS, ROWS, 128), jnp.int32),
        pltpu.VMEM((BITS, ROWS, 128), jnp.float32),
        pltpu.SemaphoreType.DMA,
    ],
)
def _sc_gather(sample_hbm, table_hbm, out_hbm, samp_v, idx_v, gat_v, sem):
    wid = lax.axis_index("s") * NC + lax.axis_index("c")
    base_e = wid * EPW

    pltpu.sync_copy(sample_hbm.at[pl.ds(base_e, EPW)], samp_v)

    lane = lax.broadcasted_iota(jnp.int32, (L,), 0)

    def build(i, _):
        s = samp_v[pl.ds(i * L, L)]
        q = (s * 256.0).astype(jnp.int32)
        e0 = base_e + i * L
        b = e0 // HW
        hw0 = e0 - b * HW
        base_flat = b * (NODES * HW) + hw0 + lane
        r = i // 8
        col = (i % 8) * L
        for x in range(BITS):
            node = (2 ** x - 1) + lax.shift_right_logical(q, 8 - x)
            idx = base_flat + node * HW
            idx_v[x, r, pl.ds(col, L)] = idx
        return 0

    lax.fori_loop(0, VPW, build, 0)

    for x in range(BITS):
        def fire(r, _, x=x):
            pltpu.async_copy(table_hbm.at[idx_v.at[x, r]], gat_v.at[x, r], sem)
            return 0
        lax.fori_loop(0, ROWS, fire, 0)
    for x in range(BITS):
        def drain(r, _, x=x):
            pltpu.make_async_copy(
                table_hbm.at[idx_v.at[x, r]], gat_v.at[x, r], sem).wait()
            return 0
        lax.fori_loop(0, ROWS, drain, 0)

    for x in range(BITS):
        pltpu.sync_copy(gat_v.at[x], out_hbm.at[x, wid])


def _tc_body(g_ref, s_ref, o_ref):
    s = s_ref[...]
    q = (s * 256.0).astype(jnp.int32)
    total = jnp.zeros((B, HW), jnp.float32)
    for x in range(BITS):
        l = g_ref[x]
        v = ((q >> (7 - x)) & 1).astype(jnp.float32)
        sp = jnp.maximum(l, 0.0) + jnp.log1p(jnp.exp(-jnp.abs(l)))
        total = total + v * l - sp
    o_ref[...] = jnp.sum(total, axis=1)


def kernel(sample, params):
    s_flat = sample.reshape(E)
    table = params.reshape(B * NODES * HW)
    g = _sc_gather(s_flat, table)
    g = g.reshape(BITS, B, HW)
    out = pl.pallas_call(
        _tc_body,
        out_shape=jax.ShapeDtypeStruct((B,), jnp.float32),
    )(g, sample.reshape(B, HW))
    return out
